# baseline (speedup 1.0000x reference)
import functools

import jax
import jax.numpy as jnp
from jax import lax
from jax.experimental import pallas as pl
from jax.experimental.pallas import tpu as pltpu
from jax.experimental.pallas import tpu_sc as plsc

_R = 4
_U = 8


def _sc_kernel(B, S, D):
    info = plsc.get_sparse_core_info()
    NW = info.num_cores * info.num_subcores
    L = info.num_lanes
    s_per_w = S // NW
    nblk = s_per_w // _R
    ncol = D // L
    mesh = plsc.VectorSubcoreMesh(core_axis_name="c", subcore_axis_name="s")

    @functools.partial(
        pl.kernel,
        mesh=mesh,
        out_type=jax.ShapeDtypeStruct((B, S, D), jnp.float32),
        scratch_types=[
            pltpu.VMEM((2, _R, D), jnp.float32),
            pltpu.VMEM((2, B, _R, D), jnp.float32),
            pltpu.VMEM((2, B, _R, D), jnp.float32),
            pltpu.SemaphoreType.DMA,
            pltpu.SemaphoreType.DMA,
            pltpu.SemaphoreType.DMA,
            pltpu.SemaphoreType.DMA,
        ],
    )
    def k(x_hbm, pe_hbm, out_hbm, pe_v, x_v, o_v, si0, si1, so0, so1):
        wid = lax.axis_index("s") * info.num_cores + lax.axis_index("c")
        s_base = wid * s_per_w
        sin = (si0, si1)
        sout = (so0, so1)

        def fire_in(blk_idx, p, sem):
            s0 = s_base + blk_idx * _R
            pltpu.async_copy(pe_hbm.at[pl.ds(s0, _R)], pe_v.at[p], sem)
            for b in range(B):
                pltpu.async_copy(x_hbm.at[b, pl.ds(s0, _R)], x_v.at[p, b], sem)

        def drain_in(p, sem):
            pltpu.make_async_copy(pe_hbm.at[pl.ds(s_base, _R)], pe_v.at[p], sem).wait()
            for b in range(B):
                pltpu.make_async_copy(
                    x_hbm.at[b, pl.ds(s_base, _R)], x_v.at[p, b], sem
                ).wait()

        def fire_out(blk_idx, p, sem):
            s0 = s_base + blk_idx * _R
            for b in range(B):
                pltpu.async_copy(o_v.at[p, b], out_hbm.at[b, pl.ds(s0, _R)], sem)

        def drain_out(p, sem):
            for b in range(B):
                pltpu.make_async_copy(
                    o_v.at[p, b], out_hbm.at[b, pl.ds(s_base, _R)], sem
                ).wait()

        def compute(p):
            def row(r, carry):
                def col(cu, carry2):
                    for u in range(_U):
                        d0 = (cu * _U + u) * L
                        pv = pe_v[p, r, pl.ds(d0, L)]
                        for b in range(B):
                            xv = x_v[p, b, r, pl.ds(d0, L)]
                            o_v[p, b, r, pl.ds(d0, L)] = jnp.where(
                                xv == 0.0, 0.0, pv
                            )
                    return carry2

                return lax.fori_loop(0, ncol // _U, col, carry)

            lax.fori_loop(0, _R, row, None)

        fire_in(0, 0, sin[0])
        fire_in(1, 1, sin[1])

        def step(j, carry):
            for p in range(2):
                blk_idx = 2 * j + p
                drain_in(p, sin[p])

                @pl.when(j >= 1)
                def _():
                    drain_out(p, sout[p])

                compute(p)
                fire_out(blk_idx, p, sout[p])

                @pl.when(blk_idx + 2 < nblk)
                def _():
                    fire_in(blk_idx + 2, p, sin[p])

            return carry

        lax.fori_loop(0, nblk // 2, step, None)
        drain_out(0, sout[0])
        drain_out(1, sout[1])

    return k


def kernel(x, position_enc):
    B, S, D = x.shape
    pe = position_enc[:S]
    return _sc_kernel(B, S, D)(x, pe)


# speedup vs baseline: 2.7744x; 2.7744x over previous
import functools

import jax
import jax.numpy as jnp
from jax import lax
from jax.experimental import pallas as pl
from jax.experimental.pallas import tpu as pltpu
from jax.experimental.pallas import tpu_sc as plsc

_R = 8
_U = 8


def _sc_kernel(B, S, D):
    info = plsc.get_sparse_core_info()
    NW = info.num_cores * info.num_subcores
    L = info.num_lanes
    s_per_w = S // NW
    nblk = s_per_w // _R
    ncol = D // L
    mesh = plsc.VectorSubcoreMesh(core_axis_name="c", subcore_axis_name="s")

    @functools.partial(
        pl.kernel,
        mesh=mesh,
        out_type=jax.ShapeDtypeStruct((B, S, D), jnp.float32),
        scratch_types=[
            pltpu.VMEM((4, _R, D), jnp.float32),
            pltpu.VMEM((2, B, _R, D), jnp.float32),
            pltpu.VMEM((_R, D), jnp.float32),
            pltpu.SemaphoreType.DMA,
            pltpu.SemaphoreType.DMA,
            pltpu.SemaphoreType.DMA,
            pltpu.SemaphoreType.DMA,
        ],
    )
    def k(x_hbm, pe_hbm, out_hbm, pe_v, x_v, o_v, si0, si1, so0, so1):
        wid = lax.axis_index("s") * info.num_cores + lax.axis_index("c")
        s_base = wid * s_per_w
        sin = (si0, si1)
        sout = (so0, so1)

        def fire_in(blk_idx, p, q, sem):
            s0 = s_base + blk_idx * _R
            pltpu.async_copy(pe_hbm.at[pl.ds(s0, _R)], pe_v.at[q], sem)
            for b in range(B):
                pltpu.async_copy(x_hbm.at[b, pl.ds(s0, _R)], x_v.at[p, b], sem)

        def drain_in(p, q, sem):
            pltpu.make_async_copy(pe_hbm.at[pl.ds(s_base, _R)], pe_v.at[q], sem).wait()
            for b in range(B):
                pltpu.make_async_copy(
                    x_hbm.at[b, pl.ds(s_base, _R)], x_v.at[p, b], sem
                ).wait()

        def drain_out(q, sem):
            for b in range(B):
                pltpu.make_async_copy(
                    pe_v.at[q], out_hbm.at[b, pl.ds(s_base, _R)], sem
                ).wait()

        def body(i, p, q, prev_fast):
            s0 = s_base + i * _R
            drain_in(p, q, sin[p])

            @pl.when(prev_fast)
            def _():
                drain_out((q + 2) % 4, sout[p])

            def scan_batch(b, acc0):
                def col(cu, acc):
                    for u in range(_U):
                        k_ = cu * _U + u
                        r, c = k_ // ncol, k_ % ncol
                        xv = x_v[p, b, r, pl.ds(c * L, L)]
                        xi = lax.bitcast_convert_type(xv, jnp.int32)
                        acc = jnp.minimum(acc, xi & jnp.int32(0x7FFFFFFF))
                    return acc

                return lax.fori_loop(0, (_R * ncol) // _U, col, acc0)

            acc = jnp.full((L,), 1, jnp.int32)
            for b in range(B):
                acc = scan_batch(b, acc)
            block_zero = acc[0] == 0
            for l in range(1, L):
                block_zero = jnp.logical_or(block_zero, acc[l] == 0)
            fast = jnp.logical_not(block_zero)

            @pl.when(fast)
            def _():
                for b in range(B):
                    pltpu.async_copy(
                        pe_v.at[q], out_hbm.at[b, pl.ds(s0, _R)], sout[p]
                    )

            @pl.when(block_zero)
            def _():
                for b in range(B):
                    def row(r, carry):
                        def col(c, carry2):
                            xv = x_v[p, b, r, pl.ds(c * L, L)]
                            pv = pe_v[q, r, pl.ds(c * L, L)]
                            o_v[r, pl.ds(c * L, L)] = jnp.where(xv == 0.0, 0.0, pv)
                            return carry2

                        return lax.fori_loop(0, ncol, col, carry)

                    lax.fori_loop(0, _R, row, None)
                    pltpu.sync_copy(o_v, out_hbm.at[b, pl.ds(s0, _R)])

            return fast

        def step(j, carry):
            fA, fB = carry
            flags = [fA, fB]
            for p_ in range(4):
                i = 4 * j + p_
                p = p_ % 2
                fast = body(i, p, p_, flags[p_])
                flags.append(fast)
                if p_ < 2:
                    fire_in(i + 2, p, (p_ + 2) % 4, sin[p])
                else:

                    @pl.when(j < nblk // 4 - 1)
                    def _():
                        fire_in(i + 2, p, (p_ + 2) % 4, sin[p])

            return flags[4], flags[5]

        fire_in(0, 0, 0, sin[0])
        fire_in(1, 1, 1, sin[1])
        f = jnp.bool_(False)
        fA, fB = lax.fori_loop(0, nblk // 4, step, (f, f))

        @pl.when(fA)
        def _():
            drain_out((nblk - 2) % 4, sout[0])

        @pl.when(fB)
        def _():
            drain_out((nblk - 1) % 4, sout[1])

    return k


def kernel(x, position_enc):
    B, S, D = x.shape
    pe = position_enc[:S]
    return _sc_kernel(B, S, D)(x, pe)
